# Initial kernel scaffold; baseline (speedup 1.0000x reference)
import jax
import jax.numpy as jnp
from jax.experimental import pallas as pl


def kernel(x, edge_index, W1, b1, W2, b2, W3, b3, Wl, bl):
    raise NotImplementedError("write your pallas kernel here")


# speedup vs baseline: 5.2326x; 5.2326x over previous
import functools

import jax
import jax.numpy as jnp
from jax import lax
from jax.experimental import pallas as pl
from jax.experimental.pallas import tpu as pltpu
from jax.experimental.pallas import tpu_sc as plsc

N = 10000
E = 320000
LANES = 128
NTILES = 16
CPT = 160
NCHUNKS = NTILES * CPT
EPAD = NCHUNKS * LANES
CPW = NCHUNKS // 32
RPT = 640
ACC_ROWS = RPT * NTILES
BLK = 1000
GRID = N // BLK

_MESH = plsc.VectorSubcoreMesh(core_axis_name="c", subcore_axis_name="s")


@functools.partial(
    pl.kernel, mesh=_MESH,
    out_type=jax.ShapeDtypeStruct((2 * ACC_ROWS, 128), jnp.float32),
    scratch_types=[
        pltpu.VMEM((LANES,), jnp.int32),
        pltpu.VMEM((LANES, 128), jnp.float32),
        pltpu.VMEM_SHARED((ACC_ROWS, 128), jnp.float32),
    ],
)
def _deg(dstf_hbm, ones_hbm, z_hbm, out, didx, ones_v, acc):
    c = lax.axis_index("c")
    s = lax.axis_index("s")
    w = s * 2 + c
    sl = pl.ds(s * RPT, RPT)
    pltpu.sync_copy(z_hbm, acc.at[sl])
    pltpu.sync_copy(ones_hbm, ones_v)
    plsc.subcore_barrier()

    def body(g, carry):
        e0 = (w * CPW + g) * LANES
        pltpu.sync_copy(dstf_hbm.at[pl.ds(e0, LANES)], didx)
        pltpu.sync_copy(ones_v, acc.at[didx], add=True)
        return carry
    lax.fori_loop(0, CPW, body, 0)

    plsc.subcore_barrier()
    pltpu.sync_copy(acc.at[sl], out.at[pl.ds(c * ACC_ROWS + s * RPT, RPT)])


@functools.partial(
    pl.kernel, mesh=_MESH,
    out_type=jax.ShapeDtypeStruct((2 * ACC_ROWS, 128), jnp.float32),
    scratch_types=[
        pltpu.VMEM((LANES,), jnp.int32),
        pltpu.VMEM((LANES,), jnp.int32),
        pltpu.VMEM((LANES, 128), jnp.float32),
        pltpu.VMEM_SHARED((ACC_ROWS, 128), jnp.float32),
    ],
)
def _agg(srcf_hbm, dstf_hbm, t_hbm, z_hbm, out, sidx, didx, rows, acc):
    c = lax.axis_index("c")
    s = lax.axis_index("s")
    w = s * 2 + c
    sl = pl.ds(s * RPT, RPT)
    pltpu.sync_copy(z_hbm, acc.at[sl])
    plsc.subcore_barrier()

    def body(g, carry):
        e0 = (w * CPW + g) * LANES
        pltpu.sync_copy(srcf_hbm.at[pl.ds(e0, LANES)], sidx)
        pltpu.sync_copy(dstf_hbm.at[pl.ds(e0, LANES)], didx)
        pltpu.sync_copy(t_hbm.at[sidx], rows)
        pltpu.sync_copy(rows, acc.at[didx], add=True)
        return carry
    lax.fori_loop(0, CPW, body, 0)

    plsc.subcore_barrier()
    pltpu.sync_copy(acc.at[sl], out.at[pl.ds(c * ACC_ROWS + s * RPT, RPT)])


@functools.partial(
    pl.kernel, mesh=_MESH,
    out_type=jax.ShapeDtypeStruct((2 * ACC_ROWS, 128), jnp.float32),
    scratch_types=[
        pltpu.VMEM((LANES,), jnp.int32),
        pltpu.VMEM((LANES,), jnp.int32),
        pltpu.VMEM((LANES, 128), jnp.float32),
        pltpu.VMEM_SHARED((ACC_ROWS, 128), jnp.float32),
    ],
)
def _agg2(src2f_hbm, dstf_hbm, t2_hbm, z_hbm, out, sidx, didx, rows, acc):
    c = lax.axis_index("c")
    s = lax.axis_index("s")
    sl = pl.ds(s * RPT, RPT)
    pltpu.sync_copy(z_hbm, acc.at[sl])
    plsc.subcore_barrier()

    def body(g, carry):
        e0 = (s * CPT + g) * LANES
        pltpu.sync_copy(src2f_hbm.at[pl.ds(c * EPAD + e0, LANES)], sidx)
        pltpu.sync_copy(dstf_hbm.at[pl.ds(e0, LANES)], didx)
        pltpu.sync_copy(t2_hbm.at[sidx], rows)
        pltpu.sync_copy(rows, acc.at[didx], add=True)
        return carry
    lax.fori_loop(0, CPT, body, 0)

    plsc.subcore_barrier()
    pltpu.sync_copy(acc.at[sl], out.at[pl.ds(c * ACC_ROWS + s * RPT, RPT)])


def _spec(shape, blocked=True):
    if blocked:
        return pl.BlockSpec(shape, lambda i: (i, 0))
    return pl.BlockSpec(shape, lambda i: (0, 0))


def _ht_spec():
    return pl.BlockSpec((2, BLK, 128), lambda i: (0, i, 0))


def _t0(d0, d1, x):
    def body(d0r, d1r, xr, o):
        deg = d0r[:, 0:1] + d1r[:, 0:1]
        dis = lax.rsqrt(deg + 1.0)
        o[0] = jnp.broadcast_to(dis, (BLK, 128))
        o[1] = xr[...] * dis

    return pl.pallas_call(
        body, grid=(GRID,),
        in_specs=[_spec((BLK, 128)), _spec((BLK, 128)), _spec((BLK, 128))],
        out_specs=_ht_spec(),
        out_shape=jax.ShapeDtypeStruct((2, N, 128), jnp.float32),
    )(d0, d1, x)


def _t1(disc, p0, p1, xt, W, b):
    def body(dr, p0r, p1r, xtr, Wr, br, o):
        dis = dr[:, 0:1]
        g = (p0r[...] + p1r[...] + xtr[...]) * dis
        h = jnp.dot(g, Wr[...], precision=lax.Precision.HIGHEST,
                    preferred_element_type=jnp.float32) + br[...]
        hd = jnp.maximum(h, 0.0) * dis
        o[0] = hd[:, :128]
        o[1] = hd[:, 128:]

    return pl.pallas_call(
        body, grid=(GRID,),
        in_specs=[_spec((BLK, 128)), _spec((BLK, 128)), _spec((BLK, 128)),
                  _spec((BLK, 128)),
                  _spec((128, 256), blocked=False), _spec((1, 256), blocked=False)],
        out_specs=_ht_spec(),
        out_shape=jax.ShapeDtypeStruct((2, N, 128), jnp.float32),
    )(disc, p0, p1, xt, W, b)


def _tmid(disc, a0, a1, y0, y1, W, b):
    def body(dr, a0r, a1r, y0r, y1r, Wr, br, oh, og):
        dis = dr[:, 0:1]
        g = jnp.concatenate([a0r[...] + y0r[...], a1r[...] + y1r[...]],
                            axis=1) * dis
        h = jnp.dot(g, Wr[...], precision=lax.Precision.HIGHEST,
                    preferred_element_type=jnp.float32) + br[...]
        hd = jnp.maximum(h, 0.0) * dis
        oh[0] = hd[:, :128]
        oh[1] = hd[:, 128:]
        og[...] = g

    return pl.pallas_call(
        body, grid=(GRID,),
        in_specs=[_spec((BLK, 128)), _spec((BLK, 128)), _spec((BLK, 128)),
                  _spec((BLK, 128)), _spec((BLK, 128)),
                  _spec((256, 256), blocked=False), _spec((1, 256), blocked=False)],
        out_specs=(_ht_spec(), _spec((BLK, 256))),
        out_shape=(jax.ShapeDtypeStruct((2, N, 128), jnp.float32),
                   jax.ShapeDtypeStruct((N, 256), jnp.float32)),
    )(disc, a0, a1, y0, y1, W, b)


def _t4(disc, a0, a1, y0, y1, g1, g2, Wl, bl):
    def body(dr, a0r, a1r, y0r, y1r, g1r, g2r, Wr, br, o):
        dis = dr[:, 0:1]
        g3 = jnp.concatenate([a0r[...] + y0r[...], a1r[...] + y1r[...]],
                             axis=1) * dis
        gall = jnp.concatenate([g1r[...], g2r[...], g3], axis=1)
        o[...] = jnp.dot(gall, Wr[...], precision=lax.Precision.HIGHEST,
                         preferred_element_type=jnp.float32) + br[...]

    return pl.pallas_call(
        body, grid=(GRID,),
        in_specs=[_spec((BLK, 128)), _spec((BLK, 128)), _spec((BLK, 128)),
                  _spec((BLK, 128)), _spec((BLK, 128)),
                  _spec((BLK, 256)), _spec((BLK, 256)),
                  _spec((768, 128), blocked=False), _spec((1, 128), blocked=False)],
        out_specs=_spec((BLK, 128)),
        out_shape=jax.ShapeDtypeStruct((N, 128), jnp.float32),
    )(disc, a0, a1, y0, y1, g1, g2, Wl, bl)


def kernel(x, edge_index, W1, b1, W2, b2, W3, b3, Wl, bl):
    src = edge_index[0].astype(jnp.int32)
    dst = edge_index[1].astype(jnp.int32)
    pad = EPAD - E
    srcf = jnp.concatenate([src, jnp.zeros((pad,), jnp.int32)])
    dstf = jnp.concatenate([dst, N + (jnp.arange(pad, dtype=jnp.int32) % 16)])
    src2f = jnp.concatenate([srcf, srcf + N])
    ones128 = jnp.ones((LANES, 128), jnp.float32)
    z128 = jnp.zeros((RPT, 128), jnp.float32)

    dp = _deg(dstf, ones128, z128)
    t0 = _t0(dp[:N], dp[ACC_ROWS:ACC_ROWS + N], x)
    disc, xt = t0[0], t0[1]
    px = _agg(srcf, dstf, xt, z128)
    h1t = _t1(disc, px[:N], px[ACC_ROWS:ACC_ROWS + N], xt,
              W1, b1.reshape(1, -1))
    a1 = _agg2(src2f, dstf, h1t.reshape(2 * N, 128), z128)
    h2t, g1 = _tmid(disc, a1[:N], a1[ACC_ROWS:ACC_ROWS + N],
                    h1t[0], h1t[1], W2, b2.reshape(1, -1))
    a2 = _agg2(src2f, dstf, h2t.reshape(2 * N, 128), z128)
    h3t, g2 = _tmid(disc, a2[:N], a2[ACC_ROWS:ACC_ROWS + N],
                    h2t[0], h2t[1], W3, b3.reshape(1, -1))
    a3 = _agg2(src2f, dstf, h3t.reshape(2 * N, 128), z128)
    return _t4(disc, a3[:N], a3[ACC_ROWS:ACC_ROWS + N], h3t[0], h3t[1],
               g1, g2, Wl, bl.reshape(1, -1))


# speedup vs baseline: 5.3691x; 1.0261x over previous
import functools

import jax
import jax.numpy as jnp
from jax import lax
from jax.experimental import pallas as pl
from jax.experimental.pallas import tpu as pltpu
from jax.experimental.pallas import tpu_sc as plsc

N = 10000
E = 320000
LANES = 128
NTILES = 16
CPT = 160
NCHUNKS = NTILES * CPT
EPAD = NCHUNKS * LANES
CPW = NCHUNKS // 32
GRP = 8
RPT = 640
ACC_ROWS = RPT * NTILES
BLK = 1000
GRID = N // BLK

_MESH = plsc.VectorSubcoreMesh(core_axis_name="c", subcore_axis_name="s")


@functools.partial(
    pl.kernel, mesh=_MESH,
    out_type=jax.ShapeDtypeStruct((2 * ACC_ROWS, 128), jnp.float32),
    scratch_types=[
        pltpu.VMEM((LANES,), jnp.int32),
        pltpu.VMEM((LANES, 128), jnp.float32),
        pltpu.VMEM_SHARED((ACC_ROWS, 128), jnp.float32),
    ],
)
def _deg(dstf_hbm, ones_hbm, z_hbm, out, didx, ones_v, acc):
    c = lax.axis_index("c")
    s = lax.axis_index("s")
    w = s * 2 + c
    sl = pl.ds(s * RPT, RPT)
    pltpu.sync_copy(z_hbm, acc.at[sl])
    pltpu.sync_copy(ones_hbm, ones_v)
    plsc.subcore_barrier()

    def body(g, carry):
        pltpu.sync_copy(dstf_hbm.at[pl.ds((w * CPW + g) * LANES, LANES)], didx)
        pltpu.sync_copy(ones_v, acc.at[didx], add=True)
        return carry
    lax.fori_loop(0, CPW, body, 0)

    plsc.subcore_barrier()
    pltpu.sync_copy(acc.at[sl], out.at[pl.ds(c * ACC_ROWS + s * RPT, RPT)])


@functools.partial(
    pl.kernel, mesh=_MESH,
    out_type=jax.ShapeDtypeStruct((2 * ACC_ROWS, 128), jnp.float32),
    scratch_types=[
        pltpu.VMEM((GRP, LANES), jnp.int32),
        pltpu.VMEM((LANES,), jnp.int32),
        pltpu.VMEM((LANES, 128), jnp.float32),
        pltpu.VMEM_SHARED((ACC_ROWS, 128), jnp.float32),
    ],
)
def _agg(src_hbm, dstf_hbm, t_hbm, z_hbm, out, sidx8, didx, rows, acc):
    c = lax.axis_index("c")
    s = lax.axis_index("s")
    w = s * 2 + c
    sl = pl.ds(s * RPT, RPT)
    pltpu.sync_copy(z_hbm, acc.at[sl])
    plsc.subcore_barrier()

    def body(g, carry):
        ch = w * CPW + g * GRP
        pltpu.sync_copy(src_hbm.at[pl.ds(ch, GRP)], sidx8)
        for k in range(GRP):
            pltpu.sync_copy(dstf_hbm.at[pl.ds((ch + k) * LANES, LANES)], didx)
            pltpu.sync_copy(t_hbm.at[sidx8.at[k]], rows)
            pltpu.sync_copy(rows, acc.at[didx], add=True)
        return carry
    lax.fori_loop(0, CPW // GRP, body, 0)

    plsc.subcore_barrier()
    pltpu.sync_copy(acc.at[sl], out.at[pl.ds(c * ACC_ROWS + s * RPT, RPT)])


@functools.partial(
    pl.kernel, mesh=_MESH,
    out_type=jax.ShapeDtypeStruct((2 * ACC_ROWS, 128), jnp.float32),
    scratch_types=[
        pltpu.VMEM((GRP, LANES), jnp.int32),
        pltpu.VMEM((LANES,), jnp.int32),
        pltpu.VMEM((LANES, 128), jnp.float32),
        pltpu.VMEM_SHARED((ACC_ROWS, 128), jnp.float32),
    ],
)
def _agg2(src2_hbm, dstf_hbm, t2_hbm, z_hbm, out, sidx8, didx, rows, acc):
    c = lax.axis_index("c")
    s = lax.axis_index("s")
    sl = pl.ds(s * RPT, RPT)
    pltpu.sync_copy(z_hbm, acc.at[sl])
    plsc.subcore_barrier()

    def body(g, carry):
        ch = s * CPT + g * GRP
        pltpu.sync_copy(src2_hbm.at[pl.ds(c * NCHUNKS + ch, GRP)], sidx8)
        for k in range(GRP):
            pltpu.sync_copy(dstf_hbm.at[pl.ds((ch + k) * LANES, LANES)], didx)
            pltpu.sync_copy(t2_hbm.at[sidx8.at[k]], rows)
            pltpu.sync_copy(rows, acc.at[didx], add=True)
        return carry
    lax.fori_loop(0, CPT // GRP, body, 0)

    plsc.subcore_barrier()
    pltpu.sync_copy(acc.at[sl], out.at[pl.ds(c * ACC_ROWS + s * RPT, RPT)])


def _spec(shape, blocked=True):
    if blocked:
        return pl.BlockSpec(shape, lambda i: (i, 0))
    return pl.BlockSpec(shape, lambda i: (0, 0))


def _ht_spec():
    return pl.BlockSpec((2, BLK, 128), lambda i: (0, i, 0))


def _t0(d0, d1, x):
    def body(d0r, d1r, xr, o):
        deg = d0r[:, 0:1] + d1r[:, 0:1]
        dis = lax.rsqrt(deg + 1.0)
        o[0] = jnp.broadcast_to(dis, (BLK, 128))
        o[1] = xr[...] * dis

    return pl.pallas_call(
        body, grid=(GRID,),
        in_specs=[_spec((BLK, 128)), _spec((BLK, 128)), _spec((BLK, 128))],
        out_specs=_ht_spec(),
        out_shape=jax.ShapeDtypeStruct((2, N, 128), jnp.float32),
    )(d0, d1, x)


def _t1(disc, p0, p1, xt, W, b):
    def body(dr, p0r, p1r, xtr, Wr, br, o):
        dis = dr[:, 0:1]
        g = (p0r[...] + p1r[...] + xtr[...]) * dis
        h = jnp.dot(g, Wr[...], precision=lax.Precision.HIGHEST,
                    preferred_element_type=jnp.float32) + br[...]
        hd = jnp.maximum(h, 0.0) * dis
        o[0] = hd[:, :128]
        o[1] = hd[:, 128:]

    return pl.pallas_call(
        body, grid=(GRID,),
        in_specs=[_spec((BLK, 128)), _spec((BLK, 128)), _spec((BLK, 128)),
                  _spec((BLK, 128)),
                  _spec((128, 256), blocked=False), _spec((1, 256), blocked=False)],
        out_specs=_ht_spec(),
        out_shape=jax.ShapeDtypeStruct((2, N, 128), jnp.float32),
    )(disc, p0, p1, xt, W, b)


def _tmid(disc, a0, a1, y0, y1, W, b):
    def body(dr, a0r, a1r, y0r, y1r, Wr, br, oh, og):
        dis = dr[:, 0:1]
        g = jnp.concatenate([a0r[...] + y0r[...], a1r[...] + y1r[...]],
                            axis=1) * dis
        h = jnp.dot(g, Wr[...], precision=lax.Precision.HIGHEST,
                    preferred_element_type=jnp.float32) + br[...]
        hd = jnp.maximum(h, 0.0) * dis
        oh[0] = hd[:, :128]
        oh[1] = hd[:, 128:]
        og[...] = g

    return pl.pallas_call(
        body, grid=(GRID,),
        in_specs=[_spec((BLK, 128)), _spec((BLK, 128)), _spec((BLK, 128)),
                  _spec((BLK, 128)), _spec((BLK, 128)),
                  _spec((256, 256), blocked=False), _spec((1, 256), blocked=False)],
        out_specs=(_ht_spec(), _spec((BLK, 256))),
        out_shape=(jax.ShapeDtypeStruct((2, N, 128), jnp.float32),
                   jax.ShapeDtypeStruct((N, 256), jnp.float32)),
    )(disc, a0, a1, y0, y1, W, b)


def _t4(disc, a0, a1, y0, y1, g1, g2, Wl, bl):
    def body(dr, a0r, a1r, y0r, y1r, g1r, g2r, Wr, br, o):
        dis = dr[:, 0:1]
        g3 = jnp.concatenate([a0r[...] + y0r[...], a1r[...] + y1r[...]],
                             axis=1) * dis
        gall = jnp.concatenate([g1r[...], g2r[...], g3], axis=1)
        o[...] = jnp.dot(gall, Wr[...], precision=lax.Precision.HIGHEST,
                         preferred_element_type=jnp.float32) + br[...]

    return pl.pallas_call(
        body, grid=(GRID,),
        in_specs=[_spec((BLK, 128)), _spec((BLK, 128)), _spec((BLK, 128)),
                  _spec((BLK, 128)), _spec((BLK, 128)),
                  _spec((BLK, 256)), _spec((BLK, 256)),
                  _spec((768, 128), blocked=False), _spec((1, 128), blocked=False)],
        out_specs=_spec((BLK, 128)),
        out_shape=jax.ShapeDtypeStruct((N, 128), jnp.float32),
    )(disc, a0, a1, y0, y1, g1, g2, Wl, bl)


def kernel(x, edge_index, W1, b1, W2, b2, W3, b3, Wl, bl):
    src = edge_index[0].astype(jnp.int32)
    dst = edge_index[1].astype(jnp.int32)
    pad = EPAD - E
    src_p = jnp.concatenate(
        [src, jnp.zeros((pad,), jnp.int32)]).reshape(NCHUNKS, LANES)
    dstf = jnp.concatenate(
        [dst, N + (jnp.arange(pad, dtype=jnp.int32) % 16)])
    src2_p = jnp.concatenate([src_p, src_p + N], axis=0)
    ones128 = jnp.ones((LANES, 128), jnp.float32)
    z128 = jnp.zeros((RPT, 128), jnp.float32)

    dp = _deg(dstf, ones128, z128)
    t0 = _t0(dp[:N], dp[ACC_ROWS:ACC_ROWS + N], x)
    disc, xt = t0[0], t0[1]
    px = _agg(src_p, dstf, xt, z128)
    h1t = _t1(disc, px[:N], px[ACC_ROWS:ACC_ROWS + N], xt,
              W1, b1.reshape(1, -1))
    a1 = _agg2(src2_p, dstf, h1t.reshape(2 * N, 128), z128)
    h2t, g1 = _tmid(disc, a1[:N], a1[ACC_ROWS:ACC_ROWS + N],
                    h1t[0], h1t[1], W2, b2.reshape(1, -1))
    a2 = _agg2(src2_p, dstf, h2t.reshape(2 * N, 128), z128)
    h3t, g2 = _tmid(disc, a2[:N], a2[ACC_ROWS:ACC_ROWS + N],
                    h2t[0], h2t[1], W3, b3.reshape(1, -1))
    a3 = _agg2(src2_p, dstf, h3t.reshape(2 * N, 128), z128)
    return _t4(disc, a3[:N], a3[ACC_ROWS:ACC_ROWS + N], h3t[0], h3t[1],
               g1, g2, Wl, bl.reshape(1, -1))


# speedup vs baseline: 6.3765x; 1.1876x over previous
import functools

import jax
import jax.numpy as jnp
from jax import lax
from jax.experimental import pallas as pl
from jax.experimental.pallas import tpu as pltpu
from jax.experimental.pallas import tpu_sc as plsc

N = 10000
E = 320000
LANES = 128
NTILES = 16
CPT = 160
NCHUNKS = NTILES * CPT
EPAD = NCHUNKS * LANES
CPW = NCHUNKS // 32
GRP = 8
RPT = 640
ACC_ROWS = RPT * NTILES
BLK = 1000
GRID = N // BLK

_MESH = plsc.VectorSubcoreMesh(core_axis_name="c", subcore_axis_name="s")


@functools.partial(
    pl.kernel, mesh=_MESH,
    out_type=jax.ShapeDtypeStruct((2 * ACC_ROWS, 128), jnp.float32),
    scratch_types=[
        pltpu.VMEM((LANES,), jnp.int32),
        pltpu.VMEM((LANES, 128), jnp.float32),
        pltpu.VMEM_SHARED((ACC_ROWS, 128), jnp.float32),
    ],
)
def _deg(dstf_hbm, ones_hbm, z_hbm, out, didx, ones_v, acc):
    c = lax.axis_index("c")
    s = lax.axis_index("s")
    w = s * 2 + c
    sl = pl.ds(s * RPT, RPT)
    pltpu.sync_copy(z_hbm, acc.at[sl])
    pltpu.sync_copy(ones_hbm, ones_v)
    plsc.subcore_barrier()

    def body(g, carry):
        pltpu.sync_copy(dstf_hbm.at[pl.ds((w * CPW + g) * LANES, LANES)], didx)
        pltpu.sync_copy(ones_v, acc.at[didx], add=True)
        return carry
    lax.fori_loop(0, CPW, body, 0)

    plsc.subcore_barrier()
    pltpu.sync_copy(acc.at[sl], out.at[pl.ds(c * ACC_ROWS + s * RPT, RPT)])


@functools.partial(
    pl.kernel, mesh=_MESH,
    out_type=jax.ShapeDtypeStruct((2 * ACC_ROWS, 128), jnp.float32),
    scratch_types=[
        pltpu.VMEM((GRP, LANES), jnp.int32),
        pltpu.VMEM((LANES,), jnp.int32),
        pltpu.VMEM((LANES,), jnp.int32),
        pltpu.VMEM((LANES, 128), jnp.float32),
        pltpu.VMEM((LANES, 128), jnp.float32),
        pltpu.VMEM_SHARED((ACC_ROWS, 128), jnp.float32),
        pltpu.SemaphoreType.DMA,
        pltpu.SemaphoreType.DMA,
    ],
)
def _agg(src_hbm, dstf_hbm, t_hbm, z_hbm, out, sidx8, didx0, didx1,
         rows0, rows1, acc, sem0, sem1):
    c = lax.axis_index("c")
    s = lax.axis_index("s")
    w = s * 2 + c
    sl = pl.ds(s * RPT, RPT)
    pltpu.sync_copy(z_hbm, acc.at[sl])
    plsc.subcore_barrier()

    bufs = ((didx0, rows0, sem0), (didx1, rows1, sem1))

    def body(g, carry):
        ch = w * CPW + g * GRP
        pltpu.sync_copy(src_hbm.at[pl.ds(ch, GRP)], sidx8)
        cps = [None, None]
        cps[0] = pltpu.async_copy(t_hbm.at[sidx8.at[0]], rows0, sem0)
        for k in range(GRP):
            didx, rows, _ = bufs[k % 2]
            if k + 1 < GRP:
                _, rows_n, sem_n = bufs[(k + 1) % 2]
                cps[(k + 1) % 2] = pltpu.async_copy(
                    t_hbm.at[sidx8.at[k + 1]], rows_n, sem_n)
            pltpu.sync_copy(dstf_hbm.at[pl.ds((ch + k) * LANES, LANES)], didx)
            cps[k % 2].wait()
            pltpu.sync_copy(rows, acc.at[didx], add=True)
        return carry
    lax.fori_loop(0, CPW // GRP, body, 0)

    plsc.subcore_barrier()
    pltpu.sync_copy(acc.at[sl], out.at[pl.ds(c * ACC_ROWS + s * RPT, RPT)])


@functools.partial(
    pl.kernel, mesh=_MESH,
    out_type=jax.ShapeDtypeStruct((2 * ACC_ROWS, 128), jnp.float32),
    scratch_types=[
        pltpu.VMEM((GRP, LANES), jnp.int32),
        pltpu.VMEM((LANES,), jnp.int32),
        pltpu.VMEM((LANES,), jnp.int32),
        pltpu.VMEM((LANES, 128), jnp.float32),
        pltpu.VMEM((LANES, 128), jnp.float32),
        pltpu.VMEM_SHARED((ACC_ROWS, 128), jnp.float32),
        pltpu.SemaphoreType.DMA,
        pltpu.SemaphoreType.DMA,
    ],
)
def _agg2(src2_hbm, dstf_hbm, t2_hbm, z_hbm, out, sidx8, didx0, didx1,
          rows0, rows1, acc, sem0, sem1):
    c = lax.axis_index("c")
    s = lax.axis_index("s")
    sl = pl.ds(s * RPT, RPT)
    pltpu.sync_copy(z_hbm, acc.at[sl])
    plsc.subcore_barrier()

    bufs = ((didx0, rows0, sem0), (didx1, rows1, sem1))

    def body(g, carry):
        ch = s * CPT + g * GRP
        pltpu.sync_copy(src2_hbm.at[pl.ds(c * NCHUNKS + ch, GRP)], sidx8)
        cps = [None, None]
        cps[0] = pltpu.async_copy(t2_hbm.at[sidx8.at[0]], rows0, sem0)
        for k in range(GRP):
            didx, rows, _ = bufs[k % 2]
            if k + 1 < GRP:
                _, rows_n, sem_n = bufs[(k + 1) % 2]
                cps[(k + 1) % 2] = pltpu.async_copy(
                    t2_hbm.at[sidx8.at[k + 1]], rows_n, sem_n)
            pltpu.sync_copy(dstf_hbm.at[pl.ds((ch + k) * LANES, LANES)], didx)
            cps[k % 2].wait()
            pltpu.sync_copy(rows, acc.at[didx], add=True)
        return carry
    lax.fori_loop(0, CPT // GRP, body, 0)

    plsc.subcore_barrier()
    pltpu.sync_copy(acc.at[sl], out.at[pl.ds(c * ACC_ROWS + s * RPT, RPT)])


def _spec(shape, blocked=True):
    if blocked:
        return pl.BlockSpec(shape, lambda i: (i, 0))
    return pl.BlockSpec(shape, lambda i: (0, 0))


def _ht_spec():
    return pl.BlockSpec((2, BLK, 128), lambda i: (0, i, 0))


def _t0(d0, d1, x):
    def body(d0r, d1r, xr, o):
        deg = d0r[:, 0:1] + d1r[:, 0:1]
        dis = lax.rsqrt(deg + 1.0)
        o[0] = jnp.broadcast_to(dis, (BLK, 128))
        o[1] = xr[...] * dis

    return pl.pallas_call(
        body, grid=(GRID,),
        in_specs=[_spec((BLK, 128)), _spec((BLK, 128)), _spec((BLK, 128))],
        out_specs=_ht_spec(),
        out_shape=jax.ShapeDtypeStruct((2, N, 128), jnp.float32),
    )(d0, d1, x)


def _t1(disc, p0, p1, xt, W, b):
    def body(dr, p0r, p1r, xtr, Wr, br, o):
        dis = dr[:, 0:1]
        g = (p0r[...] + p1r[...] + xtr[...]) * dis
        h = jnp.dot(g, Wr[...], precision=lax.Precision.HIGHEST,
                    preferred_element_type=jnp.float32) + br[...]
        hd = jnp.maximum(h, 0.0) * dis
        o[0] = hd[:, :128]
        o[1] = hd[:, 128:]

    return pl.pallas_call(
        body, grid=(GRID,),
        in_specs=[_spec((BLK, 128)), _spec((BLK, 128)), _spec((BLK, 128)),
                  _spec((BLK, 128)),
                  _spec((128, 256), blocked=False), _spec((1, 256), blocked=False)],
        out_specs=_ht_spec(),
        out_shape=jax.ShapeDtypeStruct((2, N, 128), jnp.float32),
    )(disc, p0, p1, xt, W, b)


def _tmid(disc, a0, a1, y0, y1, W, b):
    def body(dr, a0r, a1r, y0r, y1r, Wr, br, oh, og):
        dis = dr[:, 0:1]
        g = jnp.concatenate([a0r[...] + y0r[...], a1r[...] + y1r[...]],
                            axis=1) * dis
        h = jnp.dot(g, Wr[...], precision=lax.Precision.HIGHEST,
                    preferred_element_type=jnp.float32) + br[...]
        hd = jnp.maximum(h, 0.0) * dis
        oh[0] = hd[:, :128]
        oh[1] = hd[:, 128:]
        og[...] = g

    return pl.pallas_call(
        body, grid=(GRID,),
        in_specs=[_spec((BLK, 128)), _spec((BLK, 128)), _spec((BLK, 128)),
                  _spec((BLK, 128)), _spec((BLK, 128)),
                  _spec((256, 256), blocked=False), _spec((1, 256), blocked=False)],
        out_specs=(_ht_spec(), _spec((BLK, 256))),
        out_shape=(jax.ShapeDtypeStruct((2, N, 128), jnp.float32),
                   jax.ShapeDtypeStruct((N, 256), jnp.float32)),
    )(disc, a0, a1, y0, y1, W, b)


def _t4(disc, a0, a1, y0, y1, g1, g2, Wl, bl):
    def body(dr, a0r, a1r, y0r, y1r, g1r, g2r, Wr, br, o):
        dis = dr[:, 0:1]
        g3 = jnp.concatenate([a0r[...] + y0r[...], a1r[...] + y1r[...]],
                             axis=1) * dis
        gall = jnp.concatenate([g1r[...], g2r[...], g3], axis=1)
        o[...] = jnp.dot(gall, Wr[...], precision=lax.Precision.HIGHEST,
                         preferred_element_type=jnp.float32) + br[...]

    return pl.pallas_call(
        body, grid=(GRID,),
        in_specs=[_spec((BLK, 128)), _spec((BLK, 128)), _spec((BLK, 128)),
                  _spec((BLK, 128)), _spec((BLK, 128)),
                  _spec((BLK, 256)), _spec((BLK, 256)),
                  _spec((768, 128), blocked=False), _spec((1, 128), blocked=False)],
        out_specs=_spec((BLK, 128)),
        out_shape=jax.ShapeDtypeStruct((N, 128), jnp.float32),
    )(disc, a0, a1, y0, y1, g1, g2, Wl, bl)


def kernel(x, edge_index, W1, b1, W2, b2, W3, b3, Wl, bl):
    src = edge_index[0].astype(jnp.int32)
    dst = edge_index[1].astype(jnp.int32)
    pad = EPAD - E
    src_p = jnp.concatenate(
        [src, jnp.zeros((pad,), jnp.int32)]).reshape(NCHUNKS, LANES)
    dstf = jnp.concatenate(
        [dst, N + (jnp.arange(pad, dtype=jnp.int32) % 16)])
    src2_p = jnp.concatenate([src_p, src_p + N], axis=0)
    ones128 = jnp.ones((LANES, 128), jnp.float32)
    z128 = jnp.zeros((RPT, 128), jnp.float32)

    dp = _deg(dstf, ones128, z128)
    t0 = _t0(dp[:N], dp[ACC_ROWS:ACC_ROWS + N], x)
    disc, xt = t0[0], t0[1]
    px = _agg(src_p, dstf, xt, z128)
    h1t = _t1(disc, px[:N], px[ACC_ROWS:ACC_ROWS + N], xt,
              W1, b1.reshape(1, -1))
    a1 = _agg2(src2_p, dstf, h1t.reshape(2 * N, 128), z128)
    h2t, g1 = _tmid(disc, a1[:N], a1[ACC_ROWS:ACC_ROWS + N],
                    h1t[0], h1t[1], W2, b2.reshape(1, -1))
    a2 = _agg2(src2_p, dstf, h2t.reshape(2 * N, 128), z128)
    h3t, g2 = _tmid(disc, a2[:N], a2[ACC_ROWS:ACC_ROWS + N],
                    h2t[0], h2t[1], W3, b3.reshape(1, -1))
    a3 = _agg2(src2_p, dstf, h3t.reshape(2 * N, 128), z128)
    return _t4(disc, a3[:N], a3[ACC_ROWS:ACC_ROWS + N], h3t[0], h3t[1],
               g1, g2, Wl, bl.reshape(1, -1))
